# baseline (speedup 1.0000x reference)
import jax
import jax.numpy as jnp
from jax import lax
from jax.experimental import pallas as pl
from jax.experimental.pallas import tpu as pltpu
from jax.experimental.pallas import tpu_sc as plsc

DIM = 32
CUTOFF = 4.0
N_NODES = 50000
N_EDGES = 800000

NC = 2
NS = 16
NW = NC * NS
CHUNK = 64
NCHUNKS = N_EDGES // CHUNK
KTOT = 2 * ((NCHUNKS // NW + 1 + 1) // 2)
KKMAX = KTOT // 2
N_NODES_PAD = 50048
ROWS_PER_TILE = N_NODES_PAD // NS
ZROWS = 8

EBLK = 16000
NBLK = 5000
EGRID = N_EDGES // EBLK
NGRID = N_NODES // NBLK


def _rbf_ece_body(bl_ref, wce_ref, ece_ref):
    d = bl_ref[...]
    c = lax.broadcasted_iota(jnp.int32, (EBLK, DIM), 1).astype(jnp.float32) * (CUTOFF / (DIM - 1))
    e0 = jnp.exp(-((d - c) ** 2) * 4.0)
    ec = jnp.dot(e0, wce_ref[...], preferred_element_type=jnp.float32)
    ece_ref[...] = jnp.concatenate([ec, e0], axis=1)


def _rbf_ece(bond_len, wce):
    w = 2 * DIM
    return pl.pallas_call(
        _rbf_ece_body,
        grid=(EGRID,),
        in_specs=[
            pl.BlockSpec((EBLK, 1), lambda i: (i, 0)),
            pl.BlockSpec((DIM, DIM), lambda i: (0, 0)),
        ],
        out_specs=pl.BlockSpec((EBLK, w), lambda i: (i, 0)),
        out_shape=jax.ShapeDtypeStruct((N_EDGES, w), jnp.float32),
    )(bond_len.reshape(N_EDGES, 1), wce)


def _ec_body(e_ref, wc_ref, ec_ref):
    e = e_ref[...]
    ec = jnp.dot(e, wc_ref[...], preferred_element_type=jnp.float32)
    if ec_ref.shape[1] == 2 * DIM:
        ec_ref[...] = jnp.concatenate([ec, e], axis=1)
    else:
        ec_ref[...] = ec


def _ec(e, wc, w):
    return pl.pallas_call(
        _ec_body,
        grid=(EGRID,),
        in_specs=[
            pl.BlockSpec((EBLK, DIM), lambda i: (i, 0)),
            pl.BlockSpec((DIM, DIM), lambda i: (0, 0)),
        ],
        out_specs=pl.BlockSpec((EBLK, w), lambda i: (i, 0)),
        out_shape=jax.ShapeDtypeStruct((N_EDGES, w), jnp.float32),
    )(e, wc)


def _embed_prep_body(z_ref, emb_ref, wad_ref, wb_ref, h_ref, t1_ref, t2_ref):
    z = z_ref[...]
    h = jnp.zeros((NBLK, DIM), jnp.float32)
    for s in range(5):
        h = h + jnp.where(z == s, 1.0, 0.0) * emb_ref[s:s + 1, :]
    h_ref[...] = h
    t1_ref[...] = jnp.dot(h, wad_ref[...], preferred_element_type=jnp.float32)
    t2_ref[...] = jnp.dot(h, wb_ref[...], preferred_element_type=jnp.float32)


def _embed_prep(z, emb, wad, wb):
    return pl.pallas_call(
        _embed_prep_body,
        grid=(NGRID,),
        in_specs=[
            pl.BlockSpec((NBLK, 1), lambda i: (i, 0)),
            pl.BlockSpec((5, DIM), lambda i: (0, 0)),
            pl.BlockSpec((DIM, 2 * DIM), lambda i: (0, 0)),
            pl.BlockSpec((DIM, DIM), lambda i: (0, 0)),
        ],
        out_specs=[
            pl.BlockSpec((NBLK, DIM), lambda i: (i, 0)),
            pl.BlockSpec((NBLK, 2 * DIM), lambda i: (i, 0)),
            pl.BlockSpec((NBLK, DIM), lambda i: (i, 0)),
        ],
        out_shape=[
            jax.ShapeDtypeStruct((N_NODES, DIM), jnp.float32),
            jax.ShapeDtypeStruct((N_NODES, 2 * DIM), jnp.float32),
            jax.ShapeDtypeStruct((N_NODES, DIM), jnp.float32),
        ],
    )(z.reshape(N_NODES, 1), emb, wad, wb)


def _update_prep_body(h_ref, a0_ref, a1_ref, wad_ref, wb_ref, h_out, t1_ref, t2_ref):
    agg = a0_ref[0] + a1_ref[0]
    sig = 1.0 / (1.0 + jnp.exp(-agg))
    h = h_ref[...] + agg * sig
    h_out[...] = h
    t1_ref[...] = jnp.dot(h, wad_ref[...], preferred_element_type=jnp.float32)
    t2_ref[...] = jnp.dot(h, wb_ref[...], preferred_element_type=jnp.float32)


def _update_prep(h, agg, wad, wb):
    return pl.pallas_call(
        _update_prep_body,
        grid=(NGRID,),
        in_specs=[
            pl.BlockSpec((NBLK, DIM), lambda i: (i, 0)),
            pl.BlockSpec((1, NBLK, DIM), lambda i: (0, i, 0)),
            pl.BlockSpec((1, NBLK, DIM), lambda i: (1, i, 0)),
            pl.BlockSpec((DIM, 2 * DIM), lambda i: (0, 0)),
            pl.BlockSpec((DIM, DIM), lambda i: (0, 0)),
        ],
        out_specs=[
            pl.BlockSpec((NBLK, DIM), lambda i: (i, 0)),
            pl.BlockSpec((NBLK, 2 * DIM), lambda i: (i, 0)),
            pl.BlockSpec((NBLK, DIM), lambda i: (i, 0)),
        ],
        out_shape=[
            jax.ShapeDtypeStruct((N_NODES, DIM), jnp.float32),
            jax.ShapeDtypeStruct((N_NODES, 2 * DIM), jnp.float32),
            jax.ShapeDtypeStruct((N_NODES, DIM), jnp.float32),
        ],
    )(h, agg, agg, wad, wb)


def _final_body(hl_ref, al0_ref, al1_ref, hr_ref, ar0_ref, ar1_ref,
                w1a_ref, w1b_ref, b1_ref, w2_ref, b2_ref, out_ref, acc_ref):
    i = pl.program_id(0)

    @pl.when(i == 0)
    def _():
        acc_ref[...] = jnp.zeros_like(acc_ref)

    aggl = al0_ref[0] + al1_ref[0]
    hl = hl_ref[...] + aggl * (1.0 / (1.0 + jnp.exp(-aggl)))
    aggr = ar0_ref[0] + ar1_ref[0]
    hr = hr_ref[...] + aggr * (1.0 / (1.0 + jnp.exp(-aggr)))
    acc_ref[0:1, 0:DIM] += jnp.sum(hl, axis=0, keepdims=True)
    acc_ref[1:2, 0:DIM] += jnp.sum(hr, axis=0, keepdims=True)

    @pl.when(i == NGRID - 1)
    def _():
        xl = acc_ref[0:1, 0:DIM]
        xr = acc_ref[1:2, 0:DIM]
        y = (jnp.dot(xl, w1a_ref[...], preferred_element_type=jnp.float32)
             + jnp.dot(xr, w1b_ref[...], preferred_element_type=jnp.float32)
             + b1_ref[...])
        y = jnp.where(y >= 0, y, 0.01 * y)
        out_ref[...] = jnp.dot(y, w2_ref[...], preferred_element_type=jnp.float32) + b2_ref[...]


def _final(hl, aggl, hr, aggr, w1a, w1b, b1, w2, b2):
    return pl.pallas_call(
        _final_body,
        grid=(NGRID,),
        in_specs=[
            pl.BlockSpec((NBLK, DIM), lambda i: (i, 0)),
            pl.BlockSpec((1, NBLK, DIM), lambda i: (0, i, 0)),
            pl.BlockSpec((1, NBLK, DIM), lambda i: (1, i, 0)),
            pl.BlockSpec((NBLK, DIM), lambda i: (i, 0)),
            pl.BlockSpec((1, NBLK, DIM), lambda i: (0, i, 0)),
            pl.BlockSpec((1, NBLK, DIM), lambda i: (1, i, 0)),
            pl.BlockSpec((DIM, DIM), lambda i: (0, 0)),
            pl.BlockSpec((DIM, DIM), lambda i: (0, 0)),
            pl.BlockSpec((1, DIM), lambda i: (0, 0)),
            pl.BlockSpec((DIM, 1), lambda i: (0, 0)),
            pl.BlockSpec((1, 1), lambda i: (0, 0)),
        ],
        out_specs=pl.BlockSpec((1, 1), lambda i: (0, 0)),
        out_shape=jax.ShapeDtypeStruct((1, 1), jnp.float32),
        scratch_shapes=[pltpu.VMEM((8, 128), jnp.float32)],
    )(hl, aggl, aggl, hr, aggr, aggr, w1a, w1b, b1, w2, b2)


def _sc_edge_impl(write_e, t1_hbm, t2_hbm, ece_hbm, idx_hbm, enew_hbm, agg_hbm,
                  idxs, idxd, g1, g2, ecev, env, msgv, zbuf, aggsh,
                  sw1, sw2, so):
    c = lax.axis_index("c")
    s = lax.axis_index("s")
    wid = s * NC + c

    def zrow(i, _):
        zbuf[i, pl.ds(0, 16)] = jnp.zeros((16,), jnp.float32)
        zbuf[i, pl.ds(16, 16)] = jnp.zeros((16,), jnp.float32)
        return 0
    lax.fori_loop(0, ZROWS, zrow, 0)
    base_row = s * ROWS_PER_TILE

    def zcopy(i, _):
        pltpu.sync_copy(zbuf, aggsh.at[pl.ds(base_row + i * ZROWS, ZROWS)])
        return 0
    lax.fori_loop(0, ROWS_PER_TILE // ZROWS, zcopy, 0)
    plsc.subcore_barrier()

    def issue_w1(k, b):
        cid = k * NW + wid
        pltpu.async_copy(idx_hbm.at[cid].at[0], idxs[b], sw1[b])
        pltpu.async_copy(idx_hbm.at[cid].at[1], idxd[b], sw1[b])
        pltpu.async_copy(ece_hbm.at[pl.ds(cid * CHUNK, CHUNK)], ecev[b], sw1[b])

    def drain_w1(b):
        pltpu.make_async_copy(idx_hbm.at[0].at[0], idxs[b], sw1[b]).wait()
        pltpu.make_async_copy(idx_hbm.at[0].at[1], idxd[b], sw1[b]).wait()
        pltpu.make_async_copy(ece_hbm.at[pl.ds(0, CHUNK)], ecev[b], sw1[b]).wait()

    def issue_w2(b):
        pltpu.async_copy(t1_hbm.at[idxs[b]], g1[b], sw2[b])
        pltpu.async_copy(t2_hbm.at[idxd[b]], g2[b], sw2[b])

    def drain_w2(b):
        pltpu.make_async_copy(t1_hbm.at[idxs[b]], g1[b], sw2[b]).wait()
        pltpu.make_async_copy(t2_hbm.at[idxd[b]], g2[b], sw2[b]).wait()

    def issue_o(k, b):
        cid = k * NW + wid
        if write_e:
            pltpu.async_copy(env[b], enew_hbm.at[pl.ds(cid * CHUNK, CHUNK)], so[b])
        pltpu.async_copy(msgv[b], aggsh.at[idxd[b]], so[b], add=True)

    def drain_o(b):
        if write_e:
            pltpu.make_async_copy(env[b], enew_hbm.at[pl.ds(0, CHUNK)], so[b]).wait()
        pltpu.make_async_copy(msgv[b], aggsh.at[idxd[b]], so[b]).wait()

    def compute(b):
        g1b, g2b, eceb = g1[b], g2[b], ecev[b]
        envb, msgb = env[b], msgv[b]

        @plsc.parallel_loop(0, CHUNK, 1, unroll=4)
        def edge_body(i):
            for j in range(2):
                sl = pl.ds(j * 16, 16)
                a = g1b[i, sl]
                dd = g1b[i, pl.ds(DIM + j * 16, 16)]
                b2 = g2b[i, sl]
                pre = a + b2 + eceb[i, sl]
                sig = 1.0 / (1.0 + jnp.exp(-pre))
                if write_e:
                    envb[i, sl] = eceb[i, pl.ds(DIM + j * 16, 16)] + pre * sig
                msgb[i, sl] = sig * dd

    def chunk_body(k, _):
        cid = k * NW + wid

        @pl.when(cid < NCHUNKS)
        def _():
            pltpu.sync_copy(idx_hbm.at[cid].at[0], idxs[0])
            pltpu.sync_copy(idx_hbm.at[cid].at[1], idxd[0])
            pltpu.sync_copy(ece_hbm.at[pl.ds(cid * CHUNK, CHUNK)], ecev[0])
            pltpu.sync_copy(t1_hbm.at[idxs[0]], g1[0])
            pltpu.sync_copy(t2_hbm.at[idxd[0]], g2[0])
            compute(0)
            if write_e:
                pltpu.sync_copy(env[0], enew_hbm.at[pl.ds(cid * CHUNK, CHUNK)])
            pltpu.sync_copy(msgv[0], aggsh.at[idxd[0]], add=True)
        return 0
    lax.fori_loop(0, KTOT, chunk_body, 0)

    plsc.subcore_barrier()
    pltpu.sync_copy(aggsh.at[pl.ds(base_row, ROWS_PER_TILE)],
                    agg_hbm.at[c].at[pl.ds(base_row, ROWS_PER_TILE)])


def _make_sc_edge(write_e):
    ecw = 2 * DIM if write_e else DIM
    out_type = [jax.ShapeDtypeStruct((NC, N_NODES_PAD, DIM), jnp.float32)]
    if write_e:
        out_type = [jax.ShapeDtypeStruct((N_EDGES, DIM), jnp.float32)] + out_type

    def run(write_e_, t1_hbm, t2_hbm, ece_hbm, idx_hbm, enew_hbm, agg_hbm,
            isa, isb, ida, idb, g1a, g1b, g2a, g2b, ea, eb, eva, evb, ma, mb,
            zbuf, aggsh):
        def scoped(s1a, s1b, s2a, s2b, soa, sob):
            _sc_edge_impl(write_e_, t1_hbm, t2_hbm, ece_hbm, idx_hbm,
                          enew_hbm, agg_hbm,
                          (isa, isb), (ida, idb), (g1a, g1b), (g2a, g2b),
                          (ea, eb), (eva, evb), (ma, mb), zbuf, aggsh,
                          (s1a, s1b), (s2a, s2b), (soa, sob))
        pl.run_scoped(scoped, *([pltpu.SemaphoreType.DMA] * 6))

    if write_e:
        def body(t1_hbm, t2_hbm, ece_hbm, idx_hbm, enew_hbm, agg_hbm, *scratch):
            run(True, t1_hbm, t2_hbm, ece_hbm, idx_hbm, enew_hbm, agg_hbm, *scratch)
    else:
        def body(t1_hbm, t2_hbm, ece_hbm, idx_hbm, agg_hbm, *scratch):
            run(False, t1_hbm, t2_hbm, ece_hbm, idx_hbm, None, agg_hbm, *scratch)

    return pl.kernel(
        body,
        out_type=out_type,
        mesh=plsc.VectorSubcoreMesh(core_axis_name="c", subcore_axis_name="s"),
        compiler_params=pltpu.CompilerParams(use_tc_tiling_on_sc=False),
        scratch_types=[
            pltpu.VMEM((CHUNK,), jnp.int32),
            pltpu.VMEM((CHUNK,), jnp.int32),
            pltpu.VMEM((CHUNK,), jnp.int32),
            pltpu.VMEM((CHUNK,), jnp.int32),
            pltpu.VMEM((CHUNK, 2 * DIM), jnp.float32),
            pltpu.VMEM((CHUNK, 2 * DIM), jnp.float32),
            pltpu.VMEM((CHUNK, DIM), jnp.float32),
            pltpu.VMEM((CHUNK, DIM), jnp.float32),
            pltpu.VMEM((CHUNK, ecw), jnp.float32),
            pltpu.VMEM((CHUNK, ecw), jnp.float32),
            pltpu.VMEM((CHUNK, DIM), jnp.float32),
            pltpu.VMEM((CHUNK, DIM), jnp.float32),
            pltpu.VMEM((CHUNK, DIM), jnp.float32),
            pltpu.VMEM((CHUNK, DIM), jnp.float32),
            pltpu.VMEM((ZROWS, DIM), jnp.float32),
            pltpu.VMEM_SHARED((N_NODES_PAD, DIM), jnp.float32),
        ],
    )


_sc_edge_full = _make_sc_edge(True)
_sc_edge_last = _make_sc_edge(False)


def _side(z, edge_index, bond_len, side):
    idx3 = edge_index.reshape(2, NCHUNKS, CHUNK).transpose(1, 0, 2)
    convs = side['convs']
    wads = [jnp.concatenate([cv['Wa'], cv['Wd']], axis=1) for cv in convs]

    ece1 = _rbf_ece(bond_len, convs[0]['Wc'])
    h0, t1, t2 = _embed_prep(z, side['emb'], wads[0], convs[0]['Wb'])
    e1, agg1 = _sc_edge_full(t1, t2, ece1, idx3)

    h1, t1, t2 = _update_prep(h0, agg1, wads[1], convs[1]['Wb'])
    ece2 = _ec(e1, convs[1]['Wc'], 2 * DIM)
    e2, agg2 = _sc_edge_full(t1, t2, ece2, idx3)

    h2, t1, t2 = _update_prep(h1, agg2, wads[2], convs[2]['Wb'])
    ec3 = _ec(e2, convs[2]['Wc'], DIM)
    (agg3,) = _sc_edge_last(t1, t2, ec3, idx3)
    return h2, agg3


def kernel(z_left, edge_index_left, bond_len_left, z_right, edge_index_right,
           bond_len_right, params):
    hl, aggl = _side(z_left, edge_index_left, bond_len_left, params['left'])
    hr, aggr = _side(z_right, edge_index_right, bond_len_right, params['right'])
    w1a = params['l1_w'][:DIM]
    w1b = params['l1_w'][DIM:]
    out = _final(hl, aggl, hr, aggr, w1a, w1b,
                 params['l1_b'][None, :], params['l2_w'], params['l2_b'][None, :])
    return out.reshape(1)


# speedup vs baseline: 1.6810x; 1.6810x over previous
import jax
import jax.numpy as jnp
from jax import lax
from jax.experimental import pallas as pl
from jax.experimental.pallas import tpu as pltpu
from jax.experimental.pallas import tpu_sc as plsc

DIM = 32
CUTOFF = 4.0
N_NODES = 50000
N_EDGES = 800000

NC = 2
NS = 16
NW = NC * NS
CHUNK = 64
NCHUNKS = N_EDGES // CHUNK
KTOT = 2 * ((NCHUNKS // NW + 1 + 1) // 2)
KKMAX = KTOT // 2
N_NODES_PAD = 50048
ROWS_PER_TILE = N_NODES_PAD // NS
ZROWS = 8

EBLK = 16000
NBLK = 5000
EGRID = N_EDGES // EBLK
NGRID = N_NODES // NBLK


def _rbf_ece_body(bl_ref, wce_ref, ece_ref):
    d = bl_ref[...]
    c = lax.broadcasted_iota(jnp.int32, (EBLK, DIM), 1).astype(jnp.float32) * (CUTOFF / (DIM - 1))
    e0 = jnp.exp(-((d - c) ** 2) * 4.0)
    ec = jnp.dot(e0, wce_ref[...], preferred_element_type=jnp.float32)
    ece_ref[...] = jnp.concatenate([ec, e0], axis=1)


def _rbf_ece(bond_len, wce):
    w = 2 * DIM
    return pl.pallas_call(
        _rbf_ece_body,
        grid=(EGRID,),
        in_specs=[
            pl.BlockSpec((EBLK, 1), lambda i: (i, 0)),
            pl.BlockSpec((DIM, DIM), lambda i: (0, 0)),
        ],
        out_specs=pl.BlockSpec((EBLK, w), lambda i: (i, 0)),
        out_shape=jax.ShapeDtypeStruct((N_EDGES, w), jnp.float32),
    )(bond_len.reshape(N_EDGES, 1), wce)


def _ec_body(e_ref, wc_ref, ec_ref):
    e = e_ref[...]
    ec = jnp.dot(e, wc_ref[...], preferred_element_type=jnp.float32)
    if ec_ref.shape[1] == 2 * DIM:
        ec_ref[...] = jnp.concatenate([ec, e], axis=1)
    else:
        ec_ref[...] = ec


def _ec(e, wc, w):
    return pl.pallas_call(
        _ec_body,
        grid=(EGRID,),
        in_specs=[
            pl.BlockSpec((EBLK, DIM), lambda i: (i, 0)),
            pl.BlockSpec((DIM, DIM), lambda i: (0, 0)),
        ],
        out_specs=pl.BlockSpec((EBLK, w), lambda i: (i, 0)),
        out_shape=jax.ShapeDtypeStruct((N_EDGES, w), jnp.float32),
    )(e, wc)


def _embed_prep_body(z_ref, emb_ref, wad_ref, wb_ref, h_ref, t1_ref, t2_ref):
    z = z_ref[...]
    h = jnp.zeros((NBLK, DIM), jnp.float32)
    for s in range(5):
        h = h + jnp.where(z == s, 1.0, 0.0) * emb_ref[s:s + 1, :]
    h_ref[...] = h
    t1_ref[...] = jnp.dot(h, wad_ref[...], preferred_element_type=jnp.float32)
    t2_ref[...] = jnp.dot(h, wb_ref[...], preferred_element_type=jnp.float32)


def _embed_prep(z, emb, wad, wb):
    return pl.pallas_call(
        _embed_prep_body,
        grid=(NGRID,),
        in_specs=[
            pl.BlockSpec((NBLK, 1), lambda i: (i, 0)),
            pl.BlockSpec((5, DIM), lambda i: (0, 0)),
            pl.BlockSpec((DIM, 2 * DIM), lambda i: (0, 0)),
            pl.BlockSpec((DIM, DIM), lambda i: (0, 0)),
        ],
        out_specs=[
            pl.BlockSpec((NBLK, DIM), lambda i: (i, 0)),
            pl.BlockSpec((NBLK, 2 * DIM), lambda i: (i, 0)),
            pl.BlockSpec((NBLK, DIM), lambda i: (i, 0)),
        ],
        out_shape=[
            jax.ShapeDtypeStruct((N_NODES, DIM), jnp.float32),
            jax.ShapeDtypeStruct((N_NODES, 2 * DIM), jnp.float32),
            jax.ShapeDtypeStruct((N_NODES, DIM), jnp.float32),
        ],
    )(z.reshape(N_NODES, 1), emb, wad, wb)


def _update_prep_body(h_ref, a0_ref, a1_ref, wad_ref, wb_ref, h_out, t1_ref, t2_ref):
    agg = a0_ref[0] + a1_ref[0]
    sig = 1.0 / (1.0 + jnp.exp(-agg))
    h = h_ref[...] + agg * sig
    h_out[...] = h
    t1_ref[...] = jnp.dot(h, wad_ref[...], preferred_element_type=jnp.float32)
    t2_ref[...] = jnp.dot(h, wb_ref[...], preferred_element_type=jnp.float32)


def _update_prep(h, agg, wad, wb):
    return pl.pallas_call(
        _update_prep_body,
        grid=(NGRID,),
        in_specs=[
            pl.BlockSpec((NBLK, DIM), lambda i: (i, 0)),
            pl.BlockSpec((1, NBLK, DIM), lambda i: (0, i, 0)),
            pl.BlockSpec((1, NBLK, DIM), lambda i: (1, i, 0)),
            pl.BlockSpec((DIM, 2 * DIM), lambda i: (0, 0)),
            pl.BlockSpec((DIM, DIM), lambda i: (0, 0)),
        ],
        out_specs=[
            pl.BlockSpec((NBLK, DIM), lambda i: (i, 0)),
            pl.BlockSpec((NBLK, 2 * DIM), lambda i: (i, 0)),
            pl.BlockSpec((NBLK, DIM), lambda i: (i, 0)),
        ],
        out_shape=[
            jax.ShapeDtypeStruct((N_NODES, DIM), jnp.float32),
            jax.ShapeDtypeStruct((N_NODES, 2 * DIM), jnp.float32),
            jax.ShapeDtypeStruct((N_NODES, DIM), jnp.float32),
        ],
    )(h, agg, agg, wad, wb)


def _final_body(hl_ref, al0_ref, al1_ref, hr_ref, ar0_ref, ar1_ref,
                w1a_ref, w1b_ref, b1_ref, w2_ref, b2_ref, out_ref, acc_ref):
    i = pl.program_id(0)

    @pl.when(i == 0)
    def _():
        acc_ref[...] = jnp.zeros_like(acc_ref)

    aggl = al0_ref[0] + al1_ref[0]
    hl = hl_ref[...] + aggl * (1.0 / (1.0 + jnp.exp(-aggl)))
    aggr = ar0_ref[0] + ar1_ref[0]
    hr = hr_ref[...] + aggr * (1.0 / (1.0 + jnp.exp(-aggr)))
    acc_ref[0:1, 0:DIM] += jnp.sum(hl, axis=0, keepdims=True)
    acc_ref[1:2, 0:DIM] += jnp.sum(hr, axis=0, keepdims=True)

    @pl.when(i == NGRID - 1)
    def _():
        xl = acc_ref[0:1, 0:DIM]
        xr = acc_ref[1:2, 0:DIM]
        y = (jnp.dot(xl, w1a_ref[...], preferred_element_type=jnp.float32)
             + jnp.dot(xr, w1b_ref[...], preferred_element_type=jnp.float32)
             + b1_ref[...])
        y = jnp.where(y >= 0, y, 0.01 * y)
        out_ref[...] = jnp.dot(y, w2_ref[...], preferred_element_type=jnp.float32) + b2_ref[...]


def _final(hl, aggl, hr, aggr, w1a, w1b, b1, w2, b2):
    return pl.pallas_call(
        _final_body,
        grid=(NGRID,),
        in_specs=[
            pl.BlockSpec((NBLK, DIM), lambda i: (i, 0)),
            pl.BlockSpec((1, NBLK, DIM), lambda i: (0, i, 0)),
            pl.BlockSpec((1, NBLK, DIM), lambda i: (1, i, 0)),
            pl.BlockSpec((NBLK, DIM), lambda i: (i, 0)),
            pl.BlockSpec((1, NBLK, DIM), lambda i: (0, i, 0)),
            pl.BlockSpec((1, NBLK, DIM), lambda i: (1, i, 0)),
            pl.BlockSpec((DIM, DIM), lambda i: (0, 0)),
            pl.BlockSpec((DIM, DIM), lambda i: (0, 0)),
            pl.BlockSpec((1, DIM), lambda i: (0, 0)),
            pl.BlockSpec((DIM, 1), lambda i: (0, 0)),
            pl.BlockSpec((1, 1), lambda i: (0, 0)),
        ],
        out_specs=pl.BlockSpec((1, 1), lambda i: (0, 0)),
        out_shape=jax.ShapeDtypeStruct((1, 1), jnp.float32),
        scratch_shapes=[pltpu.VMEM((8, 128), jnp.float32)],
    )(hl, aggl, aggl, hr, aggr, aggr, w1a, w1b, b1, w2, b2)


def _sc_edge_impl(write_e, t1_hbm, t2_hbm, ece_hbm, idx_hbm, enew_hbm, agg_hbm,
                  idxs, idxd, sidx, g1, g2, ecev, env, msgv, zbuf, aggsh,
                  sw1, sw2, so):
    c = lax.axis_index("c")
    s = lax.axis_index("s")
    wid = s * NC + c

    def zrow(i, _):
        zbuf[i, pl.ds(0, 16)] = jnp.zeros((16,), jnp.float32)
        zbuf[i, pl.ds(16, 16)] = jnp.zeros((16,), jnp.float32)
        return 0
    lax.fori_loop(0, ZROWS, zrow, 0)
    base_row = s * ROWS_PER_TILE

    def zcopy(i, _):
        pltpu.sync_copy(zbuf, aggsh.at[pl.ds(base_row + i * ZROWS, ZROWS)])
        return 0
    lax.fori_loop(0, ROWS_PER_TILE // ZROWS, zcopy, 0)
    plsc.subcore_barrier()

    def issue_w1(k, b):
        cid = k * NW + wid
        pltpu.async_copy(idx_hbm.at[cid].at[0], idxs[b], sw1[b])
        pltpu.async_copy(idx_hbm.at[cid].at[1], idxd[b], sw1[b])
        pltpu.async_copy(ece_hbm.at[pl.ds(cid * CHUNK, CHUNK)], ecev[b], sw1[b])

    def drain_w1(b):
        pltpu.make_async_copy(idx_hbm.at[0].at[0], idxs[b], sw1[b]).wait()
        pltpu.make_async_copy(idx_hbm.at[0].at[1], idxd[b], sw1[b]).wait()
        pltpu.make_async_copy(ece_hbm.at[pl.ds(0, CHUNK)], ecev[b], sw1[b]).wait()

    def issue_w2(b):
        pltpu.async_copy(t1_hbm.at[idxs[b]], g1[b], sw2[b])
        pltpu.async_copy(t2_hbm.at[idxd[b]], g2[b], sw2[b])

    def drain_w2(b):
        pltpu.make_async_copy(t1_hbm.at[idxs[b]], g1[b], sw2[b]).wait()
        pltpu.make_async_copy(t2_hbm.at[idxd[b]], g2[b], sw2[b]).wait()

    def issue_o(k, b):
        cid = k * NW + wid
        for i in range(CHUNK // 16):
            sidx[b][pl.ds(i * 16, 16)] = idxd[b][pl.ds(i * 16, 16)]
        if write_e:
            pltpu.async_copy(env[b], enew_hbm.at[pl.ds(cid * CHUNK, CHUNK)], so[b])
        pltpu.async_copy(msgv[b], aggsh.at[sidx[b]], so[b], add=True)

    def drain_o(b):
        if write_e:
            pltpu.make_async_copy(env[b], enew_hbm.at[pl.ds(0, CHUNK)], so[b]).wait()
        pltpu.make_async_copy(msgv[b], aggsh.at[sidx[b]], so[b]).wait()

    def compute(b):
        g1b, g2b, eceb = g1[b], g2[b], ecev[b]
        envb, msgb = env[b], msgv[b]

        @plsc.parallel_loop(0, CHUNK, 1, unroll=4)
        def edge_body(i):
            for j in range(2):
                sl = pl.ds(j * 16, 16)
                a = g1b[i, sl]
                dd = g1b[i, pl.ds(DIM + j * 16, 16)]
                b2 = g2b[i, sl]
                pre = a + b2 + eceb[i, sl]
                sig = 1.0 / (1.0 + jnp.exp(-pre))
                if write_e:
                    envb[i, sl] = eceb[i, pl.ds(DIM + j * 16, 16)] + pre * sig
                msgb[i, sl] = sig * dd

    issue_w1(0, 0)
    drain_w1(0)
    issue_w2(0)
    issue_w1(1, 1)

    def outer(kk, _):
        for b in (0, 1):
            k = kk * 2 + b
            cid = k * NW + wid

            @pl.when(cid < NCHUNKS)
            def _():
                drain_w2(b)

            @pl.when(cid + NW < NCHUNKS)
            def _():
                drain_w1(1 - b)
                issue_w2(1 - b)

            @pl.when(cid < NCHUNKS)
            def _():
                compute(b)
                if write_e:
                    pltpu.sync_copy(env[b], enew_hbm.at[pl.ds(cid * CHUNK, CHUNK)])
                pltpu.sync_copy(msgv[b], aggsh.at[idxd[b]], add=True)

            @pl.when(cid + 2 * NW < NCHUNKS)
            def _():
                issue_w1(k + 2, b)
        return 0
    lax.fori_loop(0, KKMAX, outer, 0)

    plsc.subcore_barrier()
    pltpu.sync_copy(aggsh.at[pl.ds(base_row, ROWS_PER_TILE)],
                    agg_hbm.at[c].at[pl.ds(base_row, ROWS_PER_TILE)])


def _make_sc_edge(write_e):
    ecw = 2 * DIM if write_e else DIM
    out_type = [jax.ShapeDtypeStruct((NC, N_NODES_PAD, DIM), jnp.float32)]
    if write_e:
        out_type = [jax.ShapeDtypeStruct((N_EDGES, DIM), jnp.float32)] + out_type

    def run(write_e_, t1_hbm, t2_hbm, ece_hbm, idx_hbm, enew_hbm, agg_hbm,
            isa, isb, ida, idb, sia, sib, g1a, g1b, g2a, g2b, ea, eb,
            eva, evb, ma, mb, zbuf, aggsh):
        def scoped(s1a, s1b, s2a, s2b, soa, sob):
            _sc_edge_impl(write_e_, t1_hbm, t2_hbm, ece_hbm, idx_hbm,
                          enew_hbm, agg_hbm,
                          (isa, isb), (ida, idb), (sia, sib),
                          (g1a, g1b), (g2a, g2b),
                          (ea, eb), (eva, evb), (ma, mb), zbuf, aggsh,
                          (s1a, s1b), (s2a, s2b), (soa, sob))
        pl.run_scoped(scoped, *([pltpu.SemaphoreType.DMA] * 6))

    if write_e:
        def body(t1_hbm, t2_hbm, ece_hbm, idx_hbm, enew_hbm, agg_hbm, *scratch):
            run(True, t1_hbm, t2_hbm, ece_hbm, idx_hbm, enew_hbm, agg_hbm, *scratch)
    else:
        def body(t1_hbm, t2_hbm, ece_hbm, idx_hbm, agg_hbm, *scratch):
            run(False, t1_hbm, t2_hbm, ece_hbm, idx_hbm, None, agg_hbm, *scratch)

    return pl.kernel(
        body,
        out_type=out_type,
        mesh=plsc.VectorSubcoreMesh(core_axis_name="c", subcore_axis_name="s"),
        compiler_params=pltpu.CompilerParams(use_tc_tiling_on_sc=False),
        scratch_types=[
            pltpu.VMEM((CHUNK,), jnp.int32),
            pltpu.VMEM((CHUNK,), jnp.int32),
            pltpu.VMEM((CHUNK,), jnp.int32),
            pltpu.VMEM((CHUNK,), jnp.int32),
            pltpu.VMEM((CHUNK,), jnp.int32),
            pltpu.VMEM((CHUNK,), jnp.int32),
            pltpu.VMEM((CHUNK, 2 * DIM), jnp.float32),
            pltpu.VMEM((CHUNK, 2 * DIM), jnp.float32),
            pltpu.VMEM((CHUNK, DIM), jnp.float32),
            pltpu.VMEM((CHUNK, DIM), jnp.float32),
            pltpu.VMEM((CHUNK, ecw), jnp.float32),
            pltpu.VMEM((CHUNK, ecw), jnp.float32),
            pltpu.VMEM((CHUNK, DIM), jnp.float32),
            pltpu.VMEM((CHUNK, DIM), jnp.float32),
            pltpu.VMEM((CHUNK, DIM), jnp.float32),
            pltpu.VMEM((CHUNK, DIM), jnp.float32),
            pltpu.VMEM((ZROWS, DIM), jnp.float32),
            pltpu.VMEM_SHARED((N_NODES_PAD, DIM), jnp.float32),
        ],
    )


_sc_edge_full = _make_sc_edge(True)
_sc_edge_last = _make_sc_edge(False)


def _side(z, edge_index, bond_len, side):
    idx3 = edge_index.reshape(2, NCHUNKS, CHUNK).transpose(1, 0, 2)
    convs = side['convs']
    wads = [jnp.concatenate([cv['Wa'], cv['Wd']], axis=1) for cv in convs]

    ece1 = _rbf_ece(bond_len, convs[0]['Wc'])
    h0, t1, t2 = _embed_prep(z, side['emb'], wads[0], convs[0]['Wb'])
    e1, agg1 = _sc_edge_full(t1, t2, ece1, idx3)

    h1, t1, t2 = _update_prep(h0, agg1, wads[1], convs[1]['Wb'])
    ece2 = _ec(e1, convs[1]['Wc'], 2 * DIM)
    e2, agg2 = _sc_edge_full(t1, t2, ece2, idx3)

    h2, t1, t2 = _update_prep(h1, agg2, wads[2], convs[2]['Wb'])
    ec3 = _ec(e2, convs[2]['Wc'], DIM)
    (agg3,) = _sc_edge_last(t1, t2, ec3, idx3)
    return h2, agg3


def kernel(z_left, edge_index_left, bond_len_left, z_right, edge_index_right,
           bond_len_right, params):
    hl, aggl = _side(z_left, edge_index_left, bond_len_left, params['left'])
    hr, aggr = _side(z_right, edge_index_right, bond_len_right, params['right'])
    w1a = params['l1_w'][:DIM]
    w1b = params['l1_w'][DIM:]
    out = _final(hl, aggl, hr, aggr, w1a, w1b,
                 params['l1_b'][None, :], params['l2_w'], params['l2_b'][None, :])
    return out.reshape(1)
